# Initial kernel scaffold; baseline (speedup 1.0000x reference)
import jax
import jax.numpy as jnp
from jax.experimental import pallas as pl


def kernel(x, W, b):
    raise NotImplementedError("write your pallas kernel here")


# speedup vs baseline: 1.5962x; 1.5962x over previous
import functools

import jax
import jax.numpy as jnp
from jax.experimental import pallas as pl

INPUT_DIM = 4096
NUM_EXPERTS = 64
K = 8
BLK = 1024


def _body(x_ref, w_ref, b_ref, o_ref):
    logits = jax.lax.dot_general(
        x_ref[...], w_ref[...],
        (((1,), (1,)), ((), ())),
        preferred_element_type=jnp.float32,
    ) + b_ref[...]
    m = jnp.max(logits, axis=-1, keepdims=True)
    e = jnp.exp(logits - m)
    s = jnp.sum(e, axis=-1, keepdims=True)
    rw = e / s
    cur = rw
    thresh = None
    for _ in range(K):
        thresh = jnp.max(cur, axis=-1, keepdims=True)
        cur = jnp.where(cur >= thresh, -1.0, cur)
    o_ref[...] = jnp.where(rw >= thresh, rw, 0.0)


@jax.jit
def kernel(x, W, b):
    B, S, D = x.shape
    E = W.shape[0]
    N = B * S
    x2 = x.reshape(N, D)
    out = pl.pallas_call(
        _body,
        grid=(N // BLK,),
        in_specs=[
            pl.BlockSpec((BLK, D), lambda i: (i, 0)),
            pl.BlockSpec((E, D), lambda i: (0, 0)),
            pl.BlockSpec((1, E), lambda i: (0, 0)),
        ],
        out_specs=pl.BlockSpec((BLK, E), lambda i: (i, 0)),
        out_shape=jax.ShapeDtypeStruct((N, E), jnp.float32),
    )(x2, W, b.reshape(1, E))
    return out.reshape(B, S, E)
